# baseline (speedup 1.0000x reference)
import functools

import jax
import jax.numpy as jnp
from jax import lax
from jax.experimental import pallas as pl
from jax.experimental.pallas import tpu as pltpu
from jax.experimental.pallas import tpu_sc as plsc

N_TOKENS = 32768
D_MODEL = 2048
N_EXPERTS = 8
TOP_K = 2

NC = 2
NS = 16
L = 16
NW = NC * NS
CHUNK = N_TOKENS // NW
STEPS = CHUNK // L

TOK_TILE = 512


def _matmul_body(x_ref, w_ref, b_ref, out_ref):
    out_ref[...] = (
        jnp.dot(x_ref[...], w_ref[...], preferred_element_type=jnp.float32)
        + b_ref[...]
    )


def _matmul(x, w, b2d):
    return pl.pallas_call(
        _matmul_body,
        grid=(N_TOKENS // TOK_TILE,),
        in_specs=[
            pl.BlockSpec((TOK_TILE, D_MODEL), lambda i: (i, 0)),
            pl.BlockSpec((D_MODEL, N_EXPERTS), lambda i: (0, 0)),
            pl.BlockSpec((1, N_EXPERTS), lambda i: (0, 0)),
        ],
        out_specs=pl.BlockSpec((TOK_TILE, N_EXPERTS), lambda i: (i, 0)),
        out_shape=jax.ShapeDtypeStruct((N_TOKENS, N_EXPERTS), jnp.float32),
        compiler_params=pltpu.CompilerParams(
            dimension_semantics=("arbitrary",),
        ),
    )(x, w, b2d)


def _router_body(logits_hbm, gates_hbm, scores_hbm, idxs_hbm,
                 logits_v, gates_v, scores_v, idxs_v):
    wid = lax.axis_index("s") * NC + lax.axis_index("c")
    pltpu.sync_copy(
        logits_hbm.at[pl.ds(wid * CHUNK * N_EXPERTS, CHUNK * N_EXPERTS)],
        logits_v)

    iota = lax.iota(jnp.int32, L)
    iota_e = iota * N_EXPERTS
    iota_k = iota * TOP_K

    def step(j, carry):
        lbase = iota_e + j * (L * N_EXPERTS)
        i1 = jnp.zeros((L,), jnp.int32)
        m1 = plsc.load_gather(logits_v, [lbase])
        m2 = jnp.full((L,), -jnp.inf, jnp.float32)
        i2 = jnp.zeros((L,), jnp.int32)
        for e in range(1, N_EXPERTS):
            col = jnp.full((L,), e, jnp.int32)
            v = plsc.load_gather(logits_v, [lbase + e])
            gt1 = v > m1
            gt2 = v > m2
            m2 = jnp.where(gt1, m1, jnp.where(gt2, v, m2))
            i2 = jnp.where(gt1, i1, jnp.where(gt2, col, i2))
            m1 = jnp.where(gt1, v, m1)
            i1 = jnp.where(gt1, col, i1)
        r = jnp.exp(m2 - m1)
        g1 = 1.0 / (1.0 + r)
        g2 = r * g1
        obase = iota_k + j * (L * TOP_K)
        plsc.store_scatter(scores_v, [obase], m1)
        plsc.store_scatter(scores_v, [obase + 1], m2)
        plsc.store_scatter(gates_v, [obase], g1)
        plsc.store_scatter(gates_v, [obase + 1], g2)
        plsc.store_scatter(idxs_v, [obase], i1)
        plsc.store_scatter(idxs_v, [obase + 1], i2)
        return carry

    lax.fori_loop(0, STEPS, step, 0)
    obase = wid * CHUNK * TOP_K
    pltpu.sync_copy(gates_v, gates_hbm.at[pl.ds(obase, CHUNK * TOP_K)])
    pltpu.sync_copy(scores_v, scores_hbm.at[pl.ds(obase, CHUNK * TOP_K)])
    pltpu.sync_copy(idxs_v, idxs_hbm.at[pl.ds(obase, CHUNK * TOP_K)])


_router = functools.partial(
    pl.kernel,
    out_type=(
        jax.ShapeDtypeStruct((N_TOKENS * TOP_K,), jnp.float32),
        jax.ShapeDtypeStruct((N_TOKENS * TOP_K,), jnp.float32),
        jax.ShapeDtypeStruct((N_TOKENS * TOP_K,), jnp.int32),
    ),
    mesh=plsc.VectorSubcoreMesh(
        core_axis_name="c", subcore_axis_name="s",
        num_cores=NC, num_subcores=NS,
    ),
    scratch_types=[
        pltpu.VMEM((CHUNK * N_EXPERTS,), jnp.float32),
        pltpu.VMEM((CHUNK * TOP_K,), jnp.float32),
        pltpu.VMEM((CHUNK * TOP_K,), jnp.float32),
        pltpu.VMEM((CHUNK * TOP_K,), jnp.int32),
    ],
    compiler_params=pltpu.CompilerParams(needs_layout_passes=False),
)(_router_body)


def kernel(x, W, b):
    logits = _matmul(x, W, b.reshape(1, N_EXPERTS))
    gates, top_scores, top_idxs = _router(logits.reshape(-1))
    shape = (N_TOKENS, TOP_K)
    return (gates.reshape(shape), top_scores.reshape(shape),
            top_idxs.reshape(shape))


# speedup vs baseline: 1.0841x; 1.0841x over previous
import functools

import jax
import jax.numpy as jnp
from jax import lax
from jax.experimental import pallas as pl
from jax.experimental.pallas import tpu as pltpu
from jax.experimental.pallas import tpu_sc as plsc

N_TOKENS = 32768
D_MODEL = 2048
N_EXPERTS = 8
TOP_K = 2

NC = 2
NS = 16
L = 16
NW = NC * NS
CHUNK = N_TOKENS // NW
STEPS = CHUNK // L

TOK_TILE = 1024
D_SPLIT = 4
D_CHUNK = D_MODEL // D_SPLIT


def _matmul_body(*refs):
    x_refs = refs[:D_SPLIT]
    w_ref, b_ref, out_ref = refs[D_SPLIT:]
    acc = b_ref[...]
    for c in range(D_SPLIT):
        acc = acc + jnp.dot(
            x_refs[c][...],
            w_ref[c * D_CHUNK:(c + 1) * D_CHUNK, :],
            preferred_element_type=jnp.float32,
        )
    out_ref[...] = acc


def _x_spec(c):
    return pl.BlockSpec((TOK_TILE, D_CHUNK), lambda i, c=c: (i, c))


def _matmul(x, w, b2d):
    return pl.pallas_call(
        _matmul_body,
        grid=(N_TOKENS // TOK_TILE,),
        in_specs=[
            *[_x_spec(c) for c in range(D_SPLIT)],
            pl.BlockSpec((D_MODEL, N_EXPERTS), lambda i: (0, 0)),
            pl.BlockSpec((1, N_EXPERTS), lambda i: (0, 0)),
        ],
        out_specs=pl.BlockSpec((TOK_TILE, N_EXPERTS), lambda i: (i, 0)),
        out_shape=jax.ShapeDtypeStruct((N_TOKENS, N_EXPERTS), jnp.float32),
        compiler_params=pltpu.CompilerParams(
            dimension_semantics=("arbitrary",),
        ),
    )(*([x] * D_SPLIT), w, b2d)


def _router_body(logits_hbm, gates_hbm, scores_hbm, idxs_hbm,
                 logits_v, gates_v, scores_v, idxs_v):
    wid = lax.axis_index("s") * NC + lax.axis_index("c")
    pltpu.sync_copy(
        logits_hbm.at[pl.ds(wid * CHUNK * N_EXPERTS, CHUNK * N_EXPERTS)],
        logits_v)

    iota = lax.iota(jnp.int32, L)
    iota_e = iota * N_EXPERTS
    iota_k = iota * TOP_K

    def step(j, carry):
        lbase = iota_e + j * (L * N_EXPERTS)
        i1 = jnp.zeros((L,), jnp.int32)
        m1 = plsc.load_gather(logits_v, [lbase])
        m2 = jnp.full((L,), -jnp.inf, jnp.float32)
        i2 = jnp.zeros((L,), jnp.int32)
        for e in range(1, N_EXPERTS):
            col = jnp.full((L,), e, jnp.int32)
            v = plsc.load_gather(logits_v, [lbase + e])
            gt1 = v > m1
            gt2 = v > m2
            m2 = jnp.where(gt1, m1, jnp.where(gt2, v, m2))
            i2 = jnp.where(gt1, i1, jnp.where(gt2, col, i2))
            m1 = jnp.where(gt1, v, m1)
            i1 = jnp.where(gt1, col, i1)
        r = jnp.exp(m2 - m1)
        g1 = 1.0 / (1.0 + r)
        g2 = r * g1
        obase = iota_k + j * (L * TOP_K)
        plsc.store_scatter(scores_v, [obase], m1)
        plsc.store_scatter(scores_v, [obase + 1], m2)
        plsc.store_scatter(gates_v, [obase], g1)
        plsc.store_scatter(gates_v, [obase + 1], g2)
        plsc.store_scatter(idxs_v, [obase], i1)
        plsc.store_scatter(idxs_v, [obase + 1], i2)
        return carry

    lax.fori_loop(0, STEPS, step, 0)
    obase = wid * CHUNK * TOP_K
    pltpu.sync_copy(gates_v, gates_hbm.at[pl.ds(obase, CHUNK * TOP_K)])
    pltpu.sync_copy(scores_v, scores_hbm.at[pl.ds(obase, CHUNK * TOP_K)])
    pltpu.sync_copy(idxs_v, idxs_hbm.at[pl.ds(obase, CHUNK * TOP_K)])


_router = functools.partial(
    pl.kernel,
    out_type=(
        jax.ShapeDtypeStruct((N_TOKENS * TOP_K,), jnp.float32),
        jax.ShapeDtypeStruct((N_TOKENS * TOP_K,), jnp.float32),
        jax.ShapeDtypeStruct((N_TOKENS * TOP_K,), jnp.int32),
    ),
    mesh=plsc.VectorSubcoreMesh(
        core_axis_name="c", subcore_axis_name="s",
        num_cores=NC, num_subcores=NS,
    ),
    scratch_types=[
        pltpu.VMEM((CHUNK * N_EXPERTS,), jnp.float32),
        pltpu.VMEM((CHUNK * TOP_K,), jnp.float32),
        pltpu.VMEM((CHUNK * TOP_K,), jnp.float32),
        pltpu.VMEM((CHUNK * TOP_K,), jnp.int32),
    ],
    compiler_params=pltpu.CompilerParams(needs_layout_passes=False),
)(_router_body)


def kernel(x, W, b):
    logits = _matmul(x, W, b.reshape(1, N_EXPERTS))
    gates, top_scores, top_idxs = _router(logits.reshape(-1))
    shape = (N_TOKENS, TOP_K)
    return (gates.reshape(shape), top_scores.reshape(shape),
            top_idxs.reshape(shape))
